# baseline (speedup 1.0000x reference)
import functools
import jax
import jax.numpy as jnp
import numpy as np
from jax.experimental import pallas as pl
from jax.experimental.pallas import tpu as pltpu

N = 10000
E = 160000
G = 64
D = 640
H = 8
L = 6
DH = D // H


def _transformer_conv(x, src, dst, ea, Wq, Wk, Wv, Wr, We):
    n = x.shape[0]
    q = (x @ Wq).reshape(n, H, DH)
    k = (x @ Wk).reshape(n, H, DH)
    v = (x @ Wv).reshape(n, H, DH)
    e = (ea @ We).reshape(-1, H, DH)
    ke = k[src] + e
    ve = v[src] + e
    score = (q[dst] * ke).sum(-1) / np.sqrt(DH).astype(np.float32)
    smax = jax.ops.segment_max(score, dst, num_segments=n)
    smax = jnp.where(jnp.isfinite(smax), smax, 0.0)
    ex = jnp.exp(score - smax[dst])
    den = jax.ops.segment_sum(ex, dst, num_segments=n)
    alpha = ex / (den[dst] + 1e-16)
    agg = jax.ops.segment_sum(ve * alpha[:, :, None], dst, num_segments=n)
    return agg.reshape(n, D) + x @ Wr


def _graph_layers(x, edge_index, edge_attr, params):
    (Wq0, Wk0, Wv0, Wr0, We0, WqR, WkR, WvR, WrR, WeR) = params
    src = edge_index[0]
    dst = edge_index[1]
    ea = edge_attr[:, None]
    h = x[:, 3:11]
    h = _transformer_conv(h, src, dst, ea, Wq0, Wk0, Wv0, Wr0, We0)
    h = jax.nn.relu(h)
    for l in range(L - 1):
        h = _transformer_conv(h, src, dst, ea, WqR[l], WkR[l], WvR[l], WrR[l], WeR[l])
        if l < L - 2:
            h = jax.nn.relu(h)
    return h


def _head_kernel(hi_ref, hj_ref, bi_ref, bj_ref, lng_ref, lnb_ref, lw_ref, lb_ref,
                 out_ref):
    def embed(h_ref, b_ref):
        b = b_ref[:]
        gids = jax.lax.broadcasted_iota(jnp.int32, (G, h_ref.shape[0]), 0)
        onehot = jnp.where(gids == b[None, :], 1.0, 0.0)
        pooled = jax.lax.dot_general(
            onehot, h_ref[:], (((1,), (0,)), ((), ())),
            preferred_element_type=jnp.float32)
        mu = pooled.mean(-1, keepdims=True)
        var = ((pooled - mu) ** 2).mean(-1, keepdims=True)
        z = (pooled - mu) / jnp.sqrt(var + 1e-5) * lng_ref[:][None, :] + lnb_ref[:][None, :]
        emb = jnp.maximum(
            jax.lax.dot_general(z, lw_ref[:], (((1,), (0,)), ((), ())),
                                preferred_element_type=jnp.float32)
            + lb_ref[:][None, :], 0.0)
        return emb

    emb_i = embed(hi_ref, bi_ref)
    emb_j = embed(hj_ref, bj_ref)
    dot = (emb_i * emb_j).sum(-1)
    ni = jnp.maximum(jnp.sqrt((emb_i * emb_i).sum(-1)), 1e-8)
    nj = jnp.maximum(jnp.sqrt((emb_j * emb_j).sum(-1)), 1e-8)
    out_ref[:] = dot / (ni * nj)


@jax.jit
def kernel(x_i, edge_index_i, edge_attr_i, batch_i, x_j, edge_index_j,
           edge_attr_j, batch_j, Wq0, Wk0, Wv0, Wr0, We0, WqR, WkR, WvR, WrR,
           WeR, ln_g, ln_b, lin_W, lin_b):
    params = (Wq0, Wk0, Wv0, Wr0, We0, WqR, WkR, WvR, WrR, WeR)
    h_i = _graph_layers(x_i, edge_index_i, edge_attr_i, params)
    h_j = _graph_layers(x_j, edge_index_j, edge_attr_j, params)
    out = pl.pallas_call(
        _head_kernel,
        out_shape=jax.ShapeDtypeStruct((G,), jnp.float32),
    )(h_i, h_j, batch_i.astype(jnp.int32), batch_j.astype(jnp.int32),
      ln_g, ln_b, lin_W, lin_b)
    return out


# speedup vs baseline: 5.6806x; 5.6806x over previous
import functools
import jax
import jax.numpy as jnp
import numpy as np
from jax import lax
from jax.experimental import pallas as pl
from jax.experimental.pallas import tpu as pltpu
from jax.experimental.pallas import tpu_sc as plsc

N = 10000
E = 160000
G = 64
D = 640
H = 8
L = 6
DH = D // H

_NC, _NS = 2, 16
_TILES = _NC * _NS
_NB = 32
_NPAD = 10240
_NU = 2 * _NPAD
_BPT = _NU // (_TILES * _NB)
_NBLK = _NU // _NB
_EU = 2 * E
_EPAD = 320064
_MROWS = _EPAD // 16
_RPPAD = 664
_G2 = 2 * G
_MB = 512
_GRID = _NU // _MB

_ORIGIN_NP = np.empty(D, np.int32)
for _h in range(H):
    for _j in range(DH):
        _lane = _h if _j % 2 == 0 else 15 - _h
        _ORIGIN_NP[(_j // 2) * 16 + _lane] = _h * DH + _j
_HEADEQ_NP = np.zeros((D, 16), np.float32)
for _tc in range(D):
    _hh = _ORIGIN_NP[_tc] // DH
    for _l in range(16):
        if (_l if _l < 8 else 15 - _l) == _hh:
            _HEADEQ_NP[_tc, _l] = 1.0

_PREC = jax.lax.Precision.HIGHEST


def _proj_kernel(flag_ref, agg_ref, rin_ref, w_ref,
                 q_ref, k_ref, v_ref, r_ref, qwe_ref):
    h = agg_ref[:] + rin_ref[:]
    act = jnp.where(flag_ref[0] > 0, jnp.maximum(h, 0.0), h)
    out = lax.dot_general(act, w_ref[:], (((1,), (0,)), ((), ())),
                          preferred_element_type=jnp.float32,
                          precision=_PREC)
    q_ref[:] = out[:, 0:D]
    k_ref[:] = out[:, D:2 * D]
    v_ref[:] = out[:, 2 * D:3 * D]
    r_ref[:] = out[:, 3 * D:4 * D]
    qwe_ref[:] = out[:, 4 * D:4 * D + 16]


_proj = pl.pallas_call(
    _proj_kernel, grid=(_GRID,),
    in_specs=[
        pl.BlockSpec(memory_space=pltpu.SMEM),
        pl.BlockSpec((_MB, D), lambda i: (i, 0)),
        pl.BlockSpec((_MB, D), lambda i: (i, 0)),
        pl.BlockSpec((D, 4 * D + 16), lambda i: (0, 0)),
    ],
    out_specs=[
        pl.BlockSpec((_MB, D), lambda i: (i, 0)),
        pl.BlockSpec((_MB, D), lambda i: (i, 0)),
        pl.BlockSpec((_MB, D), lambda i: (i, 0)),
        pl.BlockSpec((_MB, D), lambda i: (i, 0)),
        pl.BlockSpec((_MB, 16), lambda i: (i, 0)),
    ],
    out_shape=[
        jax.ShapeDtypeStruct((_NU, D), jnp.float32),
        jax.ShapeDtypeStruct((_NU, D), jnp.float32),
        jax.ShapeDtypeStruct((_NU, D), jnp.float32),
        jax.ShapeDtypeStruct((_NU, D), jnp.float32),
        jax.ShapeDtypeStruct((_NU, 16), jnp.float32),
    ])


def _sc_edge_body(q_hbm, k_hbm, v_hbm, qwe_hbm, meta_hbm, eam_hbm, rp_hbm,
                  wet_hbm, out_hbm,
                  meta_v, eam_v, idx_s, krows, vrows, q_blk, qwe_blk, agg_blk,
                  acc, wet_v, rp_v):
    wid = lax.axis_index("s") * _NC + lax.axis_index("c")
    pltpu.sync_copy(wet_hbm, wet_v)
    pltpu.sync_copy(rp_hbm, rp_v)

    zero16 = jnp.zeros((16,), jnp.float32)
    neg16 = jnp.full((16,), -1e30, jnp.float32)
    one16 = jnp.ones((16,), jnp.float32)

    def zacc(jj, _):
        acc[jj, :] = zero16
        return 0
    lax.fori_loop(0, 40, zacc, 0)

    def flush_into(prow, den, ae):
        inv = 1.0 / den
        wae = ae * inv

        def fj(jj, _):
            agg_blk[prow, pl.ds(jj * 16, 16)] = (
                acc[jj, :] * inv + wae * wet_v[jj, :])
            acc[jj, :] = zero16
            return 0
        lax.fori_loop(0, 40, fj, 0)

    def block_body(b, _):
        bid = wid * _BPT + b
        nb0 = bid * _NB

        def zrow(r, _):
            def zc(jj, _):
                agg_blk[r, pl.ds(jj * 16, 16)] = zero16
                return 0
            lax.fori_loop(0, 40, zc, 0)
            return 0
        lax.fori_loop(0, _NB, zrow, 0)

        pltpu.sync_copy(q_hbm.at[pl.ds(nb0, _NB)], q_blk)
        pltpu.sync_copy(qwe_hbm.at[pl.ds(nb0, _NB)], qwe_blk)
        e_lo = rp_v[pl.ds(bid, 16)][0]
        e_hi = rp_v[pl.ds(bid + 1, 16)][0]
        c0 = (e_lo // 16) * 16
        nch = (e_hi - c0 + 15) // 16

        def chunk_body(ci, carry):
            cbase = c0 + ci * 16
            pltpu.sync_copy(meta_hbm.at[pl.ds(cbase // 16, 1)], meta_v)
            pltpu.sync_copy(eam_hbm.at[pl.ds(cbase // 16, 1)], eam_v)
            idx_s[:] = meta_v[0, pl.ds(0, 16)]
            pltpu.sync_copy(k_hbm.at[idx_s], krows)
            pltpu.sync_copy(v_hbm.at[idx_s], vrows)

            def edge_body(i, c):
                m, den, ae, prev = c
                n_i = meta_v[0, pl.ds(16 + i, 16)][0]
                ea_i = eam_v[0, pl.ds(i, 16)][0]
                e_abs = cbase + i
                active = jnp.logical_and(e_abs >= e_lo, e_abs < e_hi)
                is_new = jnp.logical_and(active, n_i != prev)

                def do_flush(_):
                    flush_into(prev - nb0, den, ae)
                    return 0

                lax.cond(jnp.logical_and(is_new, prev >= 0), do_flush,
                         lambda _: 0, 0)

                m_eff = jnp.where(is_new, neg16, m)
                den_eff = jnp.where(is_new, zero16, den)
                ae_eff = jnp.where(is_new, zero16, ae)
                nrow = jnp.clip(n_i - nb0, 0, _NB - 1)

                def dj(jj, s):
                    return s + (q_blk[nrow, pl.ds(jj * 16, 16)]
                                * krows[i, pl.ds(jj * 16, 16)])
                s = lax.fori_loop(0, 40, dj, zero16)
                s = s + lax.rev(s, (0,))
                score = s + ea_i * qwe_blk[nrow, :]
                m2 = jnp.maximum(m_eff, score)
                corr = jnp.where(active, jnp.exp(m_eff - m2), one16)
                p = jnp.where(active, jnp.exp(score - m2), zero16)
                den2 = den_eff * corr + p
                ae2 = ae_eff * corr + p * ea_i
                m_next = jnp.where(active, m2, m)

                def aj(jj, _):
                    acc[jj, :] = (acc[jj, :] * corr
                                  + p * vrows[i, pl.ds(jj * 16, 16)])
                    return 0
                lax.fori_loop(0, 40, aj, 0)
                prev2 = jnp.where(active, n_i, prev)
                return (m_next, den2, ae2, prev2)

            return lax.fori_loop(0, 16, edge_body, carry)

        init = (neg16, zero16, zero16, jnp.int32(-1))
        m, den, ae, prev = lax.fori_loop(0, nch, chunk_body, init)

        def last_flush(_):
            flush_into(prev - nb0, den, ae)
            return 0
        lax.cond(prev >= 0, last_flush, lambda _: 0, 0)
        pltpu.sync_copy(agg_blk, out_hbm.at[pl.ds(nb0, _NB)])
        return 0

    lax.fori_loop(0, _BPT, block_body, 0)


_sc_edge = functools.partial(
    pl.kernel,
    out_type=jax.ShapeDtypeStruct((_NU, D), jnp.float32),
    mesh=plsc.VectorSubcoreMesh(core_axis_name="c", subcore_axis_name="s"),
    scratch_types=[
        pltpu.VMEM((1, 48), jnp.int32),
        pltpu.VMEM((1, 32), jnp.float32),
        pltpu.VMEM((16,), jnp.int32),
        pltpu.VMEM((16, D), jnp.float32),
        pltpu.VMEM((16, D), jnp.float32),
        pltpu.VMEM((_NB, D), jnp.float32),
        pltpu.VMEM((_NB, 16), jnp.float32),
        pltpu.VMEM((_NB, D), jnp.float32),
        pltpu.VMEM((40, 16), jnp.float32),
        pltpu.VMEM((40, 16), jnp.float32),
        pltpu.VMEM((_RPPAD,), jnp.int32),
    ],
)(_sc_edge_body)


def _pool_kernel(agg_ref, r_ref, oh_ref, out_ref):
    @pl.when(pl.program_id(0) == 0)
    def _():
        out_ref[:] = jnp.zeros_like(out_ref)
    h = agg_ref[:] + r_ref[:]
    out_ref[:] += lax.dot_general(oh_ref[:], h, (((1,), (0,)), ((), ())),
                                  preferred_element_type=jnp.float32,
                                  precision=_PREC)


_pool = pl.pallas_call(
    _pool_kernel, grid=(_GRID,),
    in_specs=[
        pl.BlockSpec((_MB, D), lambda i: (i, 0)),
        pl.BlockSpec((_MB, D), lambda i: (i, 0)),
        pl.BlockSpec((_G2, _MB), lambda i: (0, i)),
    ],
    out_specs=pl.BlockSpec((_G2, D), lambda i: (0, 0)),
    out_shape=jax.ShapeDtypeStruct((_G2, D), jnp.float32))


def _head_kernel(p_ref, lng_ref, lnb_ref, lw_ref, lb_ref, out_ref):
    pooled = p_ref[:]
    mu = pooled.mean(-1, keepdims=True)
    var = ((pooled - mu) ** 2).mean(-1, keepdims=True)
    z = ((pooled - mu) / jnp.sqrt(var + 1e-5) * lng_ref[:][None, :]
         + lnb_ref[:][None, :])
    emb = jnp.maximum(
        lax.dot_general(z, lw_ref[:], (((1,), (0,)), ((), ())),
                        preferred_element_type=jnp.float32,
                        precision=_PREC) + lb_ref[:][None, :], 0.0)
    emb_i = emb[:G]
    emb_j = emb[G:]
    dot = (emb_i * emb_j).sum(-1)
    ni = jnp.maximum(jnp.sqrt((emb_i * emb_i).sum(-1)), 1e-8)
    nj = jnp.maximum(jnp.sqrt((emb_j * emb_j).sum(-1)), 1e-8)
    out_ref[:] = dot / (ni * nj)


_head = pl.pallas_call(
    _head_kernel,
    out_shape=jax.ShapeDtypeStruct((G,), jnp.float32))


def _prep_edges(edge_index, edge_attr, node_off, edge_off):
    src = edge_index[0].astype(jnp.int32)
    dst = edge_index[1].astype(jnp.int32)
    order = jnp.argsort(dst)
    srcs = src[order] + node_off
    dsts = dst[order]
    eas = edge_attr[order]
    rp = jnp.searchsorted(dsts, jnp.arange(_NPAD // _NB + 1) * _NB,
                          side='left').astype(jnp.int32) + edge_off
    return srcs, dsts + node_off, eas, rp


def _prep_weights(Wq, Wk, Wv, Wr, We, origin, headeq):
    scale = np.float32(1.0 / np.sqrt(DH))
    wq_t = Wq[:, origin] * scale
    wk_t = Wk[:, origin]
    wv_t = Wv[:, origin]
    wr_t = Wr[:, origin]
    we_col = We[0][origin]
    wemask = we_col[:, None] * headeq
    wqwe = wq_t @ wemask
    wcat = jnp.concatenate([wq_t, wk_t, wv_t, wr_t, wqwe], axis=1)
    wet = we_col.reshape(40, 16)
    return wcat, wet


@jax.jit
def kernel(x_i, edge_index_i, edge_attr_i, batch_i, x_j, edge_index_j,
           edge_attr_j, batch_j, Wq0, Wk0, Wv0, Wr0, We0, WqR, WkR, WvR, WrR,
           WeR, ln_g, ln_b, lin_W, lin_b):
    origin = jnp.asarray(_ORIGIN_NP)
    headeq = jnp.asarray(_HEADEQ_NP)

    w0q = jnp.zeros((D, D), jnp.float32).at[:8].set(Wq0)
    w0k = jnp.zeros((D, D), jnp.float32).at[:8].set(Wk0)
    w0v = jnp.zeros((D, D), jnp.float32).at[:8].set(Wv0)
    w0r = jnp.zeros((D, D), jnp.float32).at[:8].set(Wr0)
    wcats = []
    wets = []
    wcat0, wet0 = _prep_weights(w0q, w0k, w0v, w0r, We0, origin, headeq)
    wcats.append(wcat0)
    wets.append(wet0)
    for l in range(L - 1):
        wc, wt = _prep_weights(WqR[l][origin, :], WkR[l][origin, :],
                               WvR[l][origin, :], WrR[l][origin, :],
                               WeR[l], origin, headeq)
        wcats.append(wc)
        wets.append(wt)
    wcats = jnp.stack(wcats)
    wets = jnp.stack(wets)
    flags = jnp.array([[0]] + [[1]] * (L - 1), jnp.int32)

    si, di, ei, rpi = _prep_edges(edge_index_i, edge_attr_i, 0, 0)
    sj, dj_, ej, rpj = _prep_edges(edge_index_j, edge_attr_j, _NPAD, E)
    srcs_u = jnp.concatenate([si, sj])
    dsts_u = jnp.concatenate([di, dj_])
    eas_u = jnp.concatenate([ei, ej])
    srcs_p = jnp.pad(srcs_u, (0, _EPAD - _EU))
    dsts_p = jnp.pad(dsts_u, (0, _EPAD - _EU))
    eas_p = jnp.pad(eas_u, (0, _EPAD - _EU))
    meta = jnp.concatenate([
        srcs_p.reshape(_MROWS, 16),
        dsts_p.reshape(_MROWS, 16),
        jnp.zeros((_MROWS, 16), jnp.int32)], axis=1)
    eam = jnp.concatenate([
        eas_p.reshape(_MROWS, 16),
        jnp.zeros((_MROWS, 16), jnp.float32)], axis=1)
    rp = jnp.concatenate([rpi[:-1], rpj])
    rp = jnp.pad(rp, (0, _RPPAD - (_NBLK + 1)))

    def one_hot(batch, off):
        b = jnp.pad(batch.astype(jnp.int32), (0, _NPAD - N),
                    constant_values=_G2 + 1)
        gids = jnp.arange(_G2, dtype=jnp.int32)
        return (gids[:, None] == (b[None, :] + off)).astype(jnp.float32)

    oh = jnp.concatenate([one_hot(batch_i, 0), one_hot(batch_j, G)], axis=1)

    act0 = jnp.zeros((_NU, D), jnp.float32)
    act0 = act0.at[:N, :8].set(x_i[:, 3:11])
    act0 = act0.at[_NPAD:_NPAD + N, :8].set(x_j[:, 3:11])
    r0 = jnp.zeros((_NU, D), jnp.float32)

    def step(carry, xs):
        agg, r = carry
        wcat, wet, flag = xs
        q, k, v, r_new, qwe = _proj(flag, agg, r, wcat)
        agg_new = _sc_edge(q, k, v, qwe, meta, eam, rp, wet)
        return (agg_new, r_new), 0

    (agg, r), _ = lax.scan(step, (act0, r0), (wcats, wets, flags))

    pooled = _pool(agg, r, oh)
    lng_t = ln_g[origin]
    lnb_t = ln_b[origin]
    lw_t = lin_W[origin, :]
    return _head(pooled, lng_t, lnb_t, lw_t, lin_b)


# speedup vs baseline: 6.7436x; 1.1871x over previous
import functools
import jax
import jax.numpy as jnp
import numpy as np
from jax import lax
from jax.experimental import pallas as pl
from jax.experimental.pallas import tpu as pltpu
from jax.experimental.pallas import tpu_sc as plsc

N = 10000
E = 160000
G = 64
D = 640
H = 8
L = 6
DH = D // H

_NC, _NS = 2, 16
_TILES = _NC * _NS
_NB = 32
_NPAD = 10240
_NU = 2 * _NPAD
_BPT = _NU // (_TILES * _NB)
_NBLK = _NU // _NB
_EU = 2 * E
_EPAD = 320064
_MROWS = _EPAD // 16
_RPPAD = 664
_G2 = 2 * G
_MB = 512
_GRID = _NU // _MB

_ORIGIN_NP = np.empty(D, np.int32)
for _h in range(H):
    for _j in range(DH):
        _lane = _h if _j % 2 == 0 else 15 - _h
        _ORIGIN_NP[(_j // 2) * 16 + _lane] = _h * DH + _j
_HEADEQ_NP = np.zeros((D, 16), np.float32)
for _tc in range(D):
    _hh = _ORIGIN_NP[_tc] // DH
    for _l in range(16):
        if (_l if _l < 8 else 15 - _l) == _hh:
            _HEADEQ_NP[_tc, _l] = 1.0

_PREC = jax.lax.Precision.HIGHEST


def _proj_kernel(flag_ref, agg_ref, rin_ref, w_ref,
                 q_ref, k_ref, v_ref, r_ref, qwe_ref):
    h = agg_ref[:] + rin_ref[:]
    act = jnp.where(flag_ref[0] > 0, jnp.maximum(h, 0.0), h)
    out = lax.dot_general(act, w_ref[:], (((1,), (0,)), ((), ())),
                          preferred_element_type=jnp.float32,
                          precision=_PREC)
    q_ref[:] = out[:, 0:D]
    k_ref[:] = out[:, D:2 * D]
    v_ref[:] = out[:, 2 * D:3 * D]
    r_ref[:] = out[:, 3 * D:4 * D]
    qwe_ref[:] = out[:, 4 * D:4 * D + 16]


_proj = pl.pallas_call(
    _proj_kernel, grid=(_GRID,),
    in_specs=[
        pl.BlockSpec(memory_space=pltpu.SMEM),
        pl.BlockSpec((_MB, D), lambda i: (i, 0)),
        pl.BlockSpec((_MB, D), lambda i: (i, 0)),
        pl.BlockSpec((D, 4 * D + 16), lambda i: (0, 0)),
    ],
    out_specs=[
        pl.BlockSpec((_MB, D), lambda i: (i, 0)),
        pl.BlockSpec((_MB, D), lambda i: (i, 0)),
        pl.BlockSpec((_MB, D), lambda i: (i, 0)),
        pl.BlockSpec((_MB, D), lambda i: (i, 0)),
        pl.BlockSpec((_MB, 16), lambda i: (i, 0)),
    ],
    out_shape=[
        jax.ShapeDtypeStruct((_NU, D), jnp.float32),
        jax.ShapeDtypeStruct((_NU, D), jnp.float32),
        jax.ShapeDtypeStruct((_NU, D), jnp.float32),
        jax.ShapeDtypeStruct((_NU, D), jnp.float32),
        jax.ShapeDtypeStruct((_NU, 16), jnp.float32),
    ])


def _sc_edge_body(q_hbm, k_hbm, v_hbm, qwe_hbm, meta_hbm, eam_hbm, rp_hbm,
                  wet_hbm, out_hbm,
                  meta_v, eam_v, idx_s, krows, vrows, q_blk, qwe_blk, agg_blk,
                  acc, wet_v, rp_v):
    wid = lax.axis_index("s") * _NC + lax.axis_index("c")
    pltpu.sync_copy(wet_hbm, wet_v)
    pltpu.sync_copy(rp_hbm, rp_v)

    zero16 = jnp.zeros((16,), jnp.float32)
    neg16 = jnp.full((16,), -1e30, jnp.float32)
    one16 = jnp.ones((16,), jnp.float32)

    def zacc(j5, _):
        for u in range(8):
            acc[j5 * 8 + u, :] = zero16
        return 0
    lax.fori_loop(0, 5, zacc, 0)

    def flush_into(prow, den, ae):
        inv = 1.0 / den
        wae = ae * inv

        def fj(j5, _):
            for u in range(8):
                jj = j5 * 8 + u
                agg_blk[prow, pl.ds(jj * 16, 16)] = (
                    acc[jj, :] * inv + wae * wet_v[jj, :])
                acc[jj, :] = zero16
            return 0
        lax.fori_loop(0, 5, fj, 0)

    def block_body(b, _):
        bid = wid * _BPT + b
        nb0 = bid * _NB

        def zrow(r, _):
            def zc(j5, _):
                for u in range(8):
                    agg_blk[r, pl.ds((j5 * 8 + u) * 16, 16)] = zero16
                return 0
            lax.fori_loop(0, 5, zc, 0)
            return 0
        lax.fori_loop(0, _NB, zrow, 0)

        pltpu.sync_copy(q_hbm.at[pl.ds(nb0, _NB)], q_blk)
        pltpu.sync_copy(qwe_hbm.at[pl.ds(nb0, _NB)], qwe_blk)
        e_lo = rp_v[pl.ds(bid, 16)][0]
        e_hi = rp_v[pl.ds(bid + 1, 16)][0]
        c0 = (e_lo // 16) * 16
        nch = (e_hi - c0 + 15) // 16

        def chunk_body(ci, carry):
            cbase = c0 + ci * 16
            pltpu.sync_copy(meta_hbm.at[pl.ds(cbase // 16, 1)], meta_v)
            pltpu.sync_copy(eam_hbm.at[pl.ds(cbase // 16, 1)], eam_v)
            idx_s[:] = meta_v[0, pl.ds(0, 16)]
            pltpu.sync_copy(k_hbm.at[idx_s], krows)
            pltpu.sync_copy(v_hbm.at[idx_s], vrows)

            def edge_body(i, c):
                m, den, ae, prev = c
                n_i = meta_v[0, pl.ds(16 + i, 16)][0]
                ea_i = eam_v[0, pl.ds(i, 16)][0]
                e_abs = cbase + i
                active = jnp.logical_and(e_abs >= e_lo, e_abs < e_hi)
                is_new = jnp.logical_and(active, n_i != prev)

                def do_flush(_):
                    flush_into(prev - nb0, den, ae)
                    return 0

                lax.cond(jnp.logical_and(is_new, prev >= 0), do_flush,
                         lambda _: 0, 0)

                m_eff = jnp.where(is_new, neg16, m)
                den_eff = jnp.where(is_new, zero16, den)
                ae_eff = jnp.where(is_new, zero16, ae)
                nrow = jnp.clip(n_i - nb0, 0, _NB - 1)

                def dj(j5, s):
                    for u in range(8):
                        jj = j5 * 8 + u
                        s = s + (q_blk[nrow, pl.ds(jj * 16, 16)]
                                 * krows[i, pl.ds(jj * 16, 16)])
                    return s
                s = lax.fori_loop(0, 5, dj, zero16)
                s = s + lax.rev(s, (0,))
                score = s + ea_i * qwe_blk[nrow, :]
                m2 = jnp.maximum(m_eff, score)
                corr = jnp.where(active, jnp.exp(m_eff - m2), one16)
                p = jnp.where(active, jnp.exp(score - m2), zero16)
                den2 = den_eff * corr + p
                ae2 = ae_eff * corr + p * ea_i
                m_next = jnp.where(active, m2, m)

                def aj(j5, _):
                    for u in range(8):
                        jj = j5 * 8 + u
                        acc[jj, :] = (acc[jj, :] * corr
                                      + p * vrows[i, pl.ds(jj * 16, 16)])
                    return 0
                lax.fori_loop(0, 5, aj, 0)
                prev2 = jnp.where(active, n_i, prev)
                return (m_next, den2, ae2, prev2)

            return lax.fori_loop(0, 16, edge_body, carry)

        init = (neg16, zero16, zero16, jnp.int32(-1))
        m, den, ae, prev = lax.fori_loop(0, nch, chunk_body, init)

        def last_flush(_):
            flush_into(prev - nb0, den, ae)
            return 0
        lax.cond(prev >= 0, last_flush, lambda _: 0, 0)
        pltpu.sync_copy(agg_blk, out_hbm.at[pl.ds(nb0, _NB)])
        return 0

    lax.fori_loop(0, _BPT, block_body, 0)


_sc_edge = functools.partial(
    pl.kernel,
    out_type=jax.ShapeDtypeStruct((_NU, D), jnp.float32),
    mesh=plsc.VectorSubcoreMesh(core_axis_name="c", subcore_axis_name="s"),
    scratch_types=[
        pltpu.VMEM((1, 48), jnp.int32),
        pltpu.VMEM((1, 32), jnp.float32),
        pltpu.VMEM((16,), jnp.int32),
        pltpu.VMEM((16, D), jnp.float32),
        pltpu.VMEM((16, D), jnp.float32),
        pltpu.VMEM((_NB, D), jnp.float32),
        pltpu.VMEM((_NB, 16), jnp.float32),
        pltpu.VMEM((_NB, D), jnp.float32),
        pltpu.VMEM((40, 16), jnp.float32),
        pltpu.VMEM((40, 16), jnp.float32),
        pltpu.VMEM((_RPPAD,), jnp.int32),
    ],
)(_sc_edge_body)


def _pool_kernel(agg_ref, r_ref, oh_ref, out_ref):
    @pl.when(pl.program_id(0) == 0)
    def _():
        out_ref[:] = jnp.zeros_like(out_ref)
    h = agg_ref[:] + r_ref[:]
    out_ref[:] += lax.dot_general(oh_ref[:], h, (((1,), (0,)), ((), ())),
                                  preferred_element_type=jnp.float32,
                                  precision=_PREC)


_pool = pl.pallas_call(
    _pool_kernel, grid=(_GRID,),
    in_specs=[
        pl.BlockSpec((_MB, D), lambda i: (i, 0)),
        pl.BlockSpec((_MB, D), lambda i: (i, 0)),
        pl.BlockSpec((_G2, _MB), lambda i: (0, i)),
    ],
    out_specs=pl.BlockSpec((_G2, D), lambda i: (0, 0)),
    out_shape=jax.ShapeDtypeStruct((_G2, D), jnp.float32))


def _head_kernel(p_ref, lng_ref, lnb_ref, lw_ref, lb_ref, out_ref):
    pooled = p_ref[:]
    mu = pooled.mean(-1, keepdims=True)
    var = ((pooled - mu) ** 2).mean(-1, keepdims=True)
    z = ((pooled - mu) / jnp.sqrt(var + 1e-5) * lng_ref[:][None, :]
         + lnb_ref[:][None, :])
    emb = jnp.maximum(
        lax.dot_general(z, lw_ref[:], (((1,), (0,)), ((), ())),
                        preferred_element_type=jnp.float32,
                        precision=_PREC) + lb_ref[:][None, :], 0.0)
    emb_i = emb[:G]
    emb_j = emb[G:]
    dot = (emb_i * emb_j).sum(-1)
    ni = jnp.maximum(jnp.sqrt((emb_i * emb_i).sum(-1)), 1e-8)
    nj = jnp.maximum(jnp.sqrt((emb_j * emb_j).sum(-1)), 1e-8)
    out_ref[:] = dot / (ni * nj)


_head = pl.pallas_call(
    _head_kernel,
    out_shape=jax.ShapeDtypeStruct((G,), jnp.float32))


def _prep_edges(edge_index, edge_attr, node_off, edge_off):
    src = edge_index[0].astype(jnp.int32)
    dst = edge_index[1].astype(jnp.int32)
    order = jnp.argsort(dst)
    srcs = src[order] + node_off
    dsts = dst[order]
    eas = edge_attr[order]
    rp = jnp.searchsorted(dsts, jnp.arange(_NPAD // _NB + 1) * _NB,
                          side='left').astype(jnp.int32) + edge_off
    return srcs, dsts + node_off, eas, rp


def _prep_weights(Wq, Wk, Wv, Wr, We, origin, headeq):
    scale = np.float32(1.0 / np.sqrt(DH))
    wq_t = Wq[:, origin] * scale
    wk_t = Wk[:, origin]
    wv_t = Wv[:, origin]
    wr_t = Wr[:, origin]
    we_col = We[0][origin]
    wemask = we_col[:, None] * headeq
    wqwe = wq_t @ wemask
    wcat = jnp.concatenate([wq_t, wk_t, wv_t, wr_t, wqwe], axis=1)
    wet = we_col.reshape(40, 16)
    return wcat, wet


@jax.jit
def kernel(x_i, edge_index_i, edge_attr_i, batch_i, x_j, edge_index_j,
           edge_attr_j, batch_j, Wq0, Wk0, Wv0, Wr0, We0, WqR, WkR, WvR, WrR,
           WeR, ln_g, ln_b, lin_W, lin_b):
    origin = jnp.asarray(_ORIGIN_NP)
    headeq = jnp.asarray(_HEADEQ_NP)

    w0q = jnp.zeros((D, D), jnp.float32).at[:8].set(Wq0)
    w0k = jnp.zeros((D, D), jnp.float32).at[:8].set(Wk0)
    w0v = jnp.zeros((D, D), jnp.float32).at[:8].set(Wv0)
    w0r = jnp.zeros((D, D), jnp.float32).at[:8].set(Wr0)
    wcats = []
    wets = []
    wcat0, wet0 = _prep_weights(w0q, w0k, w0v, w0r, We0, origin, headeq)
    wcats.append(wcat0)
    wets.append(wet0)
    for l in range(L - 1):
        wc, wt = _prep_weights(WqR[l][origin, :], WkR[l][origin, :],
                               WvR[l][origin, :], WrR[l][origin, :],
                               WeR[l], origin, headeq)
        wcats.append(wc)
        wets.append(wt)
    wcats = jnp.stack(wcats)
    wets = jnp.stack(wets)
    flags = jnp.array([[0]] + [[1]] * (L - 1), jnp.int32)

    si, di, ei, rpi = _prep_edges(edge_index_i, edge_attr_i, 0, 0)
    sj, dj_, ej, rpj = _prep_edges(edge_index_j, edge_attr_j, _NPAD, E)
    srcs_u = jnp.concatenate([si, sj])
    dsts_u = jnp.concatenate([di, dj_])
    eas_u = jnp.concatenate([ei, ej])
    srcs_p = jnp.pad(srcs_u, (0, _EPAD - _EU))
    dsts_p = jnp.pad(dsts_u, (0, _EPAD - _EU))
    eas_p = jnp.pad(eas_u, (0, _EPAD - _EU))
    meta = jnp.concatenate([
        srcs_p.reshape(_MROWS, 16),
        dsts_p.reshape(_MROWS, 16),
        jnp.zeros((_MROWS, 16), jnp.int32)], axis=1)
    eam = jnp.concatenate([
        eas_p.reshape(_MROWS, 16),
        jnp.zeros((_MROWS, 16), jnp.float32)], axis=1)
    rp = jnp.concatenate([rpi[:-1], rpj])
    rp = jnp.pad(rp, (0, _RPPAD - (_NBLK + 1)))

    def one_hot(batch, off):
        b = jnp.pad(batch.astype(jnp.int32), (0, _NPAD - N),
                    constant_values=_G2 + 1)
        gids = jnp.arange(_G2, dtype=jnp.int32)
        return (gids[:, None] == (b[None, :] + off)).astype(jnp.float32)

    oh = jnp.concatenate([one_hot(batch_i, 0), one_hot(batch_j, G)], axis=1)

    act0 = jnp.zeros((_NU, D), jnp.float32)
    act0 = act0.at[:N, :8].set(x_i[:, 3:11])
    act0 = act0.at[_NPAD:_NPAD + N, :8].set(x_j[:, 3:11])
    r0 = jnp.zeros((_NU, D), jnp.float32)

    def step(carry, xs):
        agg, r = carry
        wcat, wet, flag = xs
        q, k, v, r_new, qwe = _proj(flag, agg, r, wcat)
        agg_new = _sc_edge(q, k, v, qwe, meta, eam, rp, wet)
        return (agg_new, r_new), 0

    (agg, r), _ = lax.scan(step, (act0, r0), (wcats, wets, flags))

    pooled = _pool(agg, r, oh)
    lng_t = ln_g[origin]
    lnb_t = ln_b[origin]
    lw_t = lin_W[origin, :]
    return _head(pooled, lng_t, lnb_t, lw_t, lin_b)


# speedup vs baseline: 7.0589x; 1.0468x over previous
import functools
import jax
import jax.numpy as jnp
import numpy as np
from jax import lax
from jax.experimental import pallas as pl
from jax.experimental.pallas import tpu as pltpu
from jax.experimental.pallas import tpu_sc as plsc

N = 10000
E = 160000
G = 64
D = 640
H = 8
L = 6
DH = D // H

_NC, _NS = 2, 16
_TILES = _NC * _NS
_NB = 32
_NPAD = 10240
_NU = 2 * _NPAD
_BPT = _NU // (_TILES * _NB)
_NBLK = _NU // _NB
_EU = 2 * E
_EPAD = 320064
_MROWS = _EPAD // 16
_RPPAD = 664
_G2 = 2 * G
_MB = 512
_GRID = _NU // _MB

_ORIGIN_NP = np.empty(D, np.int32)
for _h in range(H):
    for _j in range(DH):
        _lane = _h if _j % 2 == 0 else 15 - _h
        _ORIGIN_NP[(_j // 2) * 16 + _lane] = _h * DH + _j
_HEADEQ_NP = np.zeros((D, 16), np.float32)
for _tc in range(D):
    _hh = _ORIGIN_NP[_tc] // DH
    for _l in range(16):
        if (_l if _l < 8 else 15 - _l) == _hh:
            _HEADEQ_NP[_tc, _l] = 1.0

_PREC = jax.lax.Precision.HIGHEST


def _proj_kernel(flag_ref, agg_ref, rin_ref, w_ref,
                 q_ref, k_ref, v_ref, r_ref, qwe_ref):
    h = agg_ref[:] + rin_ref[:]
    act = jnp.where(flag_ref[0] > 0, jnp.maximum(h, 0.0), h)
    out = lax.dot_general(act, w_ref[:], (((1,), (0,)), ((), ())),
                          preferred_element_type=jnp.float32,
                          precision=_PREC)
    q_ref[:] = out[:, 0:D]
    k_ref[:] = out[:, D:2 * D]
    v_ref[:] = out[:, 2 * D:3 * D]
    r_ref[:] = out[:, 3 * D:4 * D]
    qwe_ref[:] = out[:, 4 * D:4 * D + 16]


_proj = pl.pallas_call(
    _proj_kernel, grid=(_GRID,),
    in_specs=[
        pl.BlockSpec(memory_space=pltpu.SMEM),
        pl.BlockSpec((_MB, D), lambda i: (i, 0)),
        pl.BlockSpec((_MB, D), lambda i: (i, 0)),
        pl.BlockSpec((D, 4 * D + 16), lambda i: (0, 0)),
    ],
    out_specs=[
        pl.BlockSpec((_MB, D), lambda i: (i, 0)),
        pl.BlockSpec((_MB, D), lambda i: (i, 0)),
        pl.BlockSpec((_MB, D), lambda i: (i, 0)),
        pl.BlockSpec((_MB, D), lambda i: (i, 0)),
        pl.BlockSpec((_MB, 16), lambda i: (i, 0)),
    ],
    out_shape=[
        jax.ShapeDtypeStruct((_NU, D), jnp.float32),
        jax.ShapeDtypeStruct((_NU, D), jnp.float32),
        jax.ShapeDtypeStruct((_NU, D), jnp.float32),
        jax.ShapeDtypeStruct((_NU, D), jnp.float32),
        jax.ShapeDtypeStruct((_NU, 16), jnp.float32),
    ])


def _sc_edge_body(q_hbm, k_hbm, v_hbm, qwe_hbm, meta_hbm, eam_hbm, rp_hbm,
                  wet_hbm, out_hbm,
                  meta_v, eam_v, idx_s, krows, vrows, q_blk, qwe_blk, agg_blk,
                  acc, wet_v, rp_v):
    wid = lax.axis_index("s") * _NC + lax.axis_index("c")
    pltpu.sync_copy(wet_hbm, wet_v)
    pltpu.sync_copy(rp_hbm, rp_v)

    zero16 = jnp.zeros((16,), jnp.float32)
    neg16 = jnp.full((16,), -1e30, jnp.float32)
    one16 = jnp.ones((16,), jnp.float32)

    def zacc(j5, _):
        for u in range(8):
            acc[j5 * 8 + u, :] = zero16
        return 0
    lax.fori_loop(0, 5, zacc, 0)

    def flush_into(prow, den, ae):
        inv = 1.0 / den
        wae = ae * inv

        def fj(j5, _):
            for u in range(8):
                jj = j5 * 8 + u
                agg_blk[prow, pl.ds(jj * 16, 16)] = (
                    acc[jj, :] * inv + wae * wet_v[jj, :])
                acc[jj, :] = zero16
            return 0
        lax.fori_loop(0, 5, fj, 0)

    def block_body(b, _):
        bid = wid * _BPT + b
        nb0 = bid * _NB

        def zrow(r, _):
            def zc(j5, _):
                for u in range(8):
                    agg_blk[r, pl.ds((j5 * 8 + u) * 16, 16)] = zero16
                return 0
            lax.fori_loop(0, 5, zc, 0)
            return 0
        lax.fori_loop(0, _NB, zrow, 0)

        pltpu.sync_copy(q_hbm.at[pl.ds(nb0, _NB)], q_blk)
        pltpu.sync_copy(qwe_hbm.at[pl.ds(nb0, _NB)], qwe_blk)
        e_lo = rp_v[pl.ds(bid, 16)][0]
        e_hi = rp_v[pl.ds(bid + 1, 16)][0]
        c0 = (e_lo // 16) * 16
        nch = (e_hi - c0 + 15) // 16

        def chunk_body(ci, carry):
            cbase = c0 + ci * 16
            pltpu.sync_copy(meta_hbm.at[pl.ds(cbase // 16, 1)], meta_v)
            pltpu.sync_copy(eam_hbm.at[pl.ds(cbase // 16, 1)], eam_v)
            idx_s[:] = meta_v[0, pl.ds(0, 16)]
            pltpu.sync_copy(k_hbm.at[idx_s], krows)
            pltpu.sync_copy(v_hbm.at[idx_s], vrows)

            def edge_body(i, c):
                m, den, ae, prev = c
                n_i = meta_v[0, pl.ds(16 + i, 16)][0]
                ea_i = eam_v[0, pl.ds(i, 16)][0]
                e_abs = cbase + i
                active = jnp.logical_and(e_abs >= e_lo, e_abs < e_hi)
                is_new = jnp.logical_and(active, n_i != prev)

                def do_flush(_):
                    flush_into(prev - nb0, den, ae)
                    return 0

                lax.cond(jnp.logical_and(is_new, prev >= 0), do_flush,
                         lambda _: 0, 0)

                m_eff = jnp.where(is_new, neg16, m)
                den_eff = jnp.where(is_new, zero16, den)
                ae_eff = jnp.where(is_new, zero16, ae)
                nrow = jnp.clip(n_i - nb0, 0, _NB - 1)

                s = zero16
                for jj in range(40):
                    s = s + (q_blk[nrow, pl.ds(jj * 16, 16)]
                             * krows[i, pl.ds(jj * 16, 16)])
                s = s + lax.rev(s, (0,))
                score = s + ea_i * qwe_blk[nrow, :]
                m2 = jnp.maximum(m_eff, score)
                corr = jnp.where(active, jnp.exp(m_eff - m2), one16)
                p = jnp.where(active, jnp.exp(score - m2), zero16)
                den2 = den_eff * corr + p
                ae2 = ae_eff * corr + p * ea_i
                m_next = jnp.where(active, m2, m)

                for jj in range(40):
                    acc[jj, :] = (acc[jj, :] * corr
                                  + p * vrows[i, pl.ds(jj * 16, 16)])
                prev2 = jnp.where(active, n_i, prev)
                return (m_next, den2, ae2, prev2)

            return lax.fori_loop(0, 16, edge_body, carry)

        init = (neg16, zero16, zero16, jnp.int32(-1))
        m, den, ae, prev = lax.fori_loop(0, nch, chunk_body, init)

        def last_flush(_):
            flush_into(prev - nb0, den, ae)
            return 0
        lax.cond(prev >= 0, last_flush, lambda _: 0, 0)
        pltpu.sync_copy(agg_blk, out_hbm.at[pl.ds(nb0, _NB)])
        return 0

    lax.fori_loop(0, _BPT, block_body, 0)


_sc_edge = functools.partial(
    pl.kernel,
    out_type=jax.ShapeDtypeStruct((_NU, D), jnp.float32),
    mesh=plsc.VectorSubcoreMesh(core_axis_name="c", subcore_axis_name="s"),
    scratch_types=[
        pltpu.VMEM((1, 48), jnp.int32),
        pltpu.VMEM((1, 32), jnp.float32),
        pltpu.VMEM((16,), jnp.int32),
        pltpu.VMEM((16, D), jnp.float32),
        pltpu.VMEM((16, D), jnp.float32),
        pltpu.VMEM((_NB, D), jnp.float32),
        pltpu.VMEM((_NB, 16), jnp.float32),
        pltpu.VMEM((_NB, D), jnp.float32),
        pltpu.VMEM((40, 16), jnp.float32),
        pltpu.VMEM((40, 16), jnp.float32),
        pltpu.VMEM((_RPPAD,), jnp.int32),
    ],
)(_sc_edge_body)


def _pool_kernel(agg_ref, r_ref, oh_ref, out_ref):
    @pl.when(pl.program_id(0) == 0)
    def _():
        out_ref[:] = jnp.zeros_like(out_ref)
    h = agg_ref[:] + r_ref[:]
    out_ref[:] += lax.dot_general(oh_ref[:], h, (((1,), (0,)), ((), ())),
                                  preferred_element_type=jnp.float32,
                                  precision=_PREC)


_pool = pl.pallas_call(
    _pool_kernel, grid=(_GRID,),
    in_specs=[
        pl.BlockSpec((_MB, D), lambda i: (i, 0)),
        pl.BlockSpec((_MB, D), lambda i: (i, 0)),
        pl.BlockSpec((_G2, _MB), lambda i: (0, i)),
    ],
    out_specs=pl.BlockSpec((_G2, D), lambda i: (0, 0)),
    out_shape=jax.ShapeDtypeStruct((_G2, D), jnp.float32))


def _head_kernel(p_ref, lng_ref, lnb_ref, lw_ref, lb_ref, out_ref):
    pooled = p_ref[:]
    mu = pooled.mean(-1, keepdims=True)
    var = ((pooled - mu) ** 2).mean(-1, keepdims=True)
    z = ((pooled - mu) / jnp.sqrt(var + 1e-5) * lng_ref[:][None, :]
         + lnb_ref[:][None, :])
    emb = jnp.maximum(
        lax.dot_general(z, lw_ref[:], (((1,), (0,)), ((), ())),
                        preferred_element_type=jnp.float32,
                        precision=_PREC) + lb_ref[:][None, :], 0.0)
    emb_i = emb[:G]
    emb_j = emb[G:]
    dot = (emb_i * emb_j).sum(-1)
    ni = jnp.maximum(jnp.sqrt((emb_i * emb_i).sum(-1)), 1e-8)
    nj = jnp.maximum(jnp.sqrt((emb_j * emb_j).sum(-1)), 1e-8)
    out_ref[:] = dot / (ni * nj)


_head = pl.pallas_call(
    _head_kernel,
    out_shape=jax.ShapeDtypeStruct((G,), jnp.float32))


def _prep_edges(edge_index, edge_attr, node_off, edge_off):
    src = edge_index[0].astype(jnp.int32)
    dst = edge_index[1].astype(jnp.int32)
    order = jnp.argsort(dst)
    srcs = src[order] + node_off
    dsts = dst[order]
    eas = edge_attr[order]
    rp = jnp.searchsorted(dsts, jnp.arange(_NPAD // _NB + 1) * _NB,
                          side='left').astype(jnp.int32) + edge_off
    return srcs, dsts + node_off, eas, rp


def _prep_weights(Wq, Wk, Wv, Wr, We, origin, headeq):
    scale = np.float32(1.0 / np.sqrt(DH))
    wq_t = Wq[:, origin] * scale
    wk_t = Wk[:, origin]
    wv_t = Wv[:, origin]
    wr_t = Wr[:, origin]
    we_col = We[0][origin]
    wemask = we_col[:, None] * headeq
    wqwe = wq_t @ wemask
    wcat = jnp.concatenate([wq_t, wk_t, wv_t, wr_t, wqwe], axis=1)
    wet = we_col.reshape(40, 16)
    return wcat, wet


@jax.jit
def kernel(x_i, edge_index_i, edge_attr_i, batch_i, x_j, edge_index_j,
           edge_attr_j, batch_j, Wq0, Wk0, Wv0, Wr0, We0, WqR, WkR, WvR, WrR,
           WeR, ln_g, ln_b, lin_W, lin_b):
    origin = jnp.asarray(_ORIGIN_NP)
    headeq = jnp.asarray(_HEADEQ_NP)

    w0q = jnp.zeros((D, D), jnp.float32).at[:8].set(Wq0)
    w0k = jnp.zeros((D, D), jnp.float32).at[:8].set(Wk0)
    w0v = jnp.zeros((D, D), jnp.float32).at[:8].set(Wv0)
    w0r = jnp.zeros((D, D), jnp.float32).at[:8].set(Wr0)
    wcats = []
    wets = []
    wcat0, wet0 = _prep_weights(w0q, w0k, w0v, w0r, We0, origin, headeq)
    wcats.append(wcat0)
    wets.append(wet0)
    for l in range(L - 1):
        wc, wt = _prep_weights(WqR[l][origin, :], WkR[l][origin, :],
                               WvR[l][origin, :], WrR[l][origin, :],
                               WeR[l], origin, headeq)
        wcats.append(wc)
        wets.append(wt)
    wcats = jnp.stack(wcats)
    wets = jnp.stack(wets)
    flags = jnp.array([[0]] + [[1]] * (L - 1), jnp.int32)

    si, di, ei, rpi = _prep_edges(edge_index_i, edge_attr_i, 0, 0)
    sj, dj_, ej, rpj = _prep_edges(edge_index_j, edge_attr_j, _NPAD, E)
    srcs_u = jnp.concatenate([si, sj])
    dsts_u = jnp.concatenate([di, dj_])
    eas_u = jnp.concatenate([ei, ej])
    srcs_p = jnp.pad(srcs_u, (0, _EPAD - _EU))
    dsts_p = jnp.pad(dsts_u, (0, _EPAD - _EU))
    eas_p = jnp.pad(eas_u, (0, _EPAD - _EU))
    meta = jnp.concatenate([
        srcs_p.reshape(_MROWS, 16),
        dsts_p.reshape(_MROWS, 16),
        jnp.zeros((_MROWS, 16), jnp.int32)], axis=1)
    eam = jnp.concatenate([
        eas_p.reshape(_MROWS, 16),
        jnp.zeros((_MROWS, 16), jnp.float32)], axis=1)
    rp = jnp.concatenate([rpi[:-1], rpj])
    rp = jnp.pad(rp, (0, _RPPAD - (_NBLK + 1)))

    def one_hot(batch, off):
        b = jnp.pad(batch.astype(jnp.int32), (0, _NPAD - N),
                    constant_values=_G2 + 1)
        gids = jnp.arange(_G2, dtype=jnp.int32)
        return (gids[:, None] == (b[None, :] + off)).astype(jnp.float32)

    oh = jnp.concatenate([one_hot(batch_i, 0), one_hot(batch_j, G)], axis=1)

    act0 = jnp.zeros((_NU, D), jnp.float32)
    act0 = act0.at[:N, :8].set(x_i[:, 3:11])
    act0 = act0.at[_NPAD:_NPAD + N, :8].set(x_j[:, 3:11])
    r0 = jnp.zeros((_NU, D), jnp.float32)

    def step(carry, xs):
        agg, r = carry
        wcat, wet, flag = xs
        q, k, v, r_new, qwe = _proj(flag, agg, r, wcat)
        agg_new = _sc_edge(q, k, v, qwe, meta, eam, rp, wet)
        return (agg_new, r_new), 0

    (agg, r), _ = lax.scan(step, (act0, r0), (wcats, wets, flags))

    pooled = _pool(agg, r, oh)
    lng_t = ln_g[origin]
    lnb_t = ln_b[origin]
    lw_t = lin_W[origin, :]
    return _head(pooled, lng_t, lnb_t, lw_t, lin_b)


# speedup vs baseline: 8.0321x; 1.1379x over previous
import functools
import jax
import jax.numpy as jnp
import numpy as np
from jax import lax
from jax.experimental import pallas as pl
from jax.experimental.pallas import tpu as pltpu
from jax.experimental.pallas import tpu_sc as plsc

N = 10000
E = 160000
G = 64
D = 640
H = 8
L = 6
DH = D // H

_NC, _NS = 2, 16
_TILES = _NC * _NS
_NB = 32
_NPAD = 10240
_NU = 2 * _NPAD
_BPT = _NU // (_TILES * _NB)
_NBLK = _NU // _NB
_EU = 2 * E
_EPAD = 320064
_MROWS = _EPAD // 16
_RPPAD = 664
_G2 = 2 * G
_MB = 512
_GRID = _NU // _MB

_ORIGIN_NP = np.empty(D, np.int32)
for _h in range(H):
    for _j in range(DH):
        _lane = _h if _j % 2 == 0 else 15 - _h
        _ORIGIN_NP[(_j // 2) * 16 + _lane] = _h * DH + _j
_HEADEQ_NP = np.zeros((D, 16), np.float32)
for _tc in range(D):
    _hh = _ORIGIN_NP[_tc] // DH
    for _l in range(16):
        if (_l if _l < 8 else 15 - _l) == _hh:
            _HEADEQ_NP[_tc, _l] = 1.0

_PREC = jax.lax.Precision.HIGHEST


def _proj_kernel(flag_ref, agg_ref, rin_ref, w_ref,
                 q_ref, k_ref, v_ref, r_ref, qwe_ref):
    h = agg_ref[:] + rin_ref[:]
    act = jnp.where(flag_ref[0] > 0, jnp.maximum(h, 0.0), h)
    out = lax.dot_general(act, w_ref[:], (((1,), (0,)), ((), ())),
                          preferred_element_type=jnp.float32,
                          precision=_PREC)
    q_ref[:] = out[:, 0:D]
    k_ref[:] = out[:, D:2 * D]
    v_ref[:] = out[:, 2 * D:3 * D]
    r_ref[:] = out[:, 3 * D:4 * D]
    qwe_ref[:] = out[:, 4 * D:4 * D + 16]


_proj = pl.pallas_call(
    _proj_kernel, grid=(_GRID,),
    in_specs=[
        pl.BlockSpec(memory_space=pltpu.SMEM),
        pl.BlockSpec((_MB, D), lambda i: (i, 0)),
        pl.BlockSpec((_MB, D), lambda i: (i, 0)),
        pl.BlockSpec((D, 4 * D + 16), lambda i: (0, 0)),
    ],
    out_specs=[
        pl.BlockSpec((_MB, D), lambda i: (i, 0)),
        pl.BlockSpec((_MB, D), lambda i: (i, 0)),
        pl.BlockSpec((_MB, D), lambda i: (i, 0)),
        pl.BlockSpec((_MB, D), lambda i: (i, 0)),
        pl.BlockSpec((_MB, 16), lambda i: (i, 0)),
    ],
    out_shape=[
        jax.ShapeDtypeStruct((_NU, D), jnp.float32),
        jax.ShapeDtypeStruct((_NU, D), jnp.float32),
        jax.ShapeDtypeStruct((_NU, D), jnp.float32),
        jax.ShapeDtypeStruct((_NU, D), jnp.float32),
        jax.ShapeDtypeStruct((_NU, 16), jnp.float32),
    ])


def _sc_edge_body(q_hbm, k_hbm, v_hbm, qwe_hbm, meta_hbm, eam_hbm, rp_hbm,
                  wet_hbm, out_hbm,
                  meta_v, eam_v, idx_s, krows, vrows, q_blk, qwe_blk, agg_blk,
                  acc, wet_v, rp_v, semA, semB):
    wid = lax.axis_index("s") * _NC + lax.axis_index("c")
    pltpu.sync_copy(wet_hbm, wet_v)
    pltpu.sync_copy(rp_hbm, rp_v)

    zero16 = jnp.zeros((16,), jnp.float32)
    neg16 = jnp.full((16,), -1e30, jnp.float32)
    one16 = jnp.ones((16,), jnp.float32)

    def zacc(j5, _):
        for u in range(8):
            acc[j5 * 8 + u, :] = zero16
        return 0
    lax.fori_loop(0, 5, zacc, 0)

    def flush_into(prow, den, ae):
        inv = 1.0 / den
        wae = ae * inv

        def fj(j5, _):
            for u in range(8):
                jj = j5 * 8 + u
                agg_blk[prow, pl.ds(jj * 16, 16)] = (
                    acc[jj, :] * inv + wae * wet_v[jj, :])
                acc[jj, :] = zero16
            return 0
        lax.fori_loop(0, 5, fj, 0)

    def block_body(b, _):
        bid = wid * _BPT + b
        nb0 = bid * _NB

        def zrow(r, _):
            def zc(j5, _):
                for u in range(8):
                    agg_blk[r, pl.ds((j5 * 8 + u) * 16, 16)] = zero16
                return 0
            lax.fori_loop(0, 5, zc, 0)
            return 0
        lax.fori_loop(0, _NB, zrow, 0)

        pltpu.sync_copy(q_hbm.at[pl.ds(nb0, _NB)], q_blk)
        pltpu.sync_copy(qwe_hbm.at[pl.ds(nb0, _NB)], qwe_blk)
        e_lo = rp_v[pl.ds(bid, 16)][0]
        e_hi = rp_v[pl.ds(bid + 1, 16)][0]
        c0 = (e_lo // 16) * 16
        nch = (e_hi - c0 + 15) // 16

        def chunk_body(ci, carry):
            cbase = c0 + ci * 16
            cm = pltpu.async_copy(meta_hbm.at[pl.ds(cbase // 16, 1)],
                                  meta_v, semA)
            ce = pltpu.async_copy(eam_hbm.at[pl.ds(cbase // 16, 1)],
                                  eam_v, semB)
            cm.wait()
            ce.wait()
            idx_s[:] = meta_v[0, pl.ds(0, 16)]
            ck = pltpu.async_copy(k_hbm.at[idx_s], krows, semA)
            cv = pltpu.async_copy(v_hbm.at[idx_s], vrows, semB)
            ck.wait()
            cv.wait()

            def edge_body(i, c):
                m, den, ae, prev = c
                n_i = meta_v[0, pl.ds(16 + i, 16)][0]
                ea_i = eam_v[0, pl.ds(i, 16)][0]
                e_abs = cbase + i
                active = jnp.logical_and(e_abs >= e_lo, e_abs < e_hi)
                is_new = jnp.logical_and(active, n_i != prev)

                def do_flush(_):
                    flush_into(prev - nb0, den, ae)
                    return 0

                lax.cond(jnp.logical_and(is_new, prev >= 0), do_flush,
                         lambda _: 0, 0)

                m_eff = jnp.where(is_new, neg16, m)
                den_eff = jnp.where(is_new, zero16, den)
                ae_eff = jnp.where(is_new, zero16, ae)
                nrow = jnp.clip(n_i - nb0, 0, _NB - 1)

                s = zero16
                for jj in range(40):
                    s = s + (q_blk[nrow, pl.ds(jj * 16, 16)]
                             * krows[i, pl.ds(jj * 16, 16)])
                s = s + lax.rev(s, (0,))
                score = s + ea_i * qwe_blk[nrow, :]
                m2 = jnp.maximum(m_eff, score)
                corr = jnp.where(active, jnp.exp(m_eff - m2), one16)
                p = jnp.where(active, jnp.exp(score - m2), zero16)
                den2 = den_eff * corr + p
                ae2 = ae_eff * corr + p * ea_i
                m_next = jnp.where(active, m2, m)

                for jj in range(40):
                    acc[jj, :] = (acc[jj, :] * corr
                                  + p * vrows[i, pl.ds(jj * 16, 16)])
                prev2 = jnp.where(active, n_i, prev)
                return (m_next, den2, ae2, prev2)

            return lax.fori_loop(0, 16, edge_body, carry)

        init = (neg16, zero16, zero16, jnp.int32(-1))
        m, den, ae, prev = lax.fori_loop(0, nch, chunk_body, init)

        def last_flush(_):
            flush_into(prev - nb0, den, ae)
            return 0
        lax.cond(prev >= 0, last_flush, lambda _: 0, 0)
        pltpu.sync_copy(agg_blk, out_hbm.at[pl.ds(nb0, _NB)])
        return 0

    lax.fori_loop(0, _BPT, block_body, 0)


_sc_edge = functools.partial(
    pl.kernel,
    out_type=jax.ShapeDtypeStruct((_NU, D), jnp.float32),
    mesh=plsc.VectorSubcoreMesh(core_axis_name="c", subcore_axis_name="s"),
    scratch_types=[
        pltpu.VMEM((1, 48), jnp.int32),
        pltpu.VMEM((1, 32), jnp.float32),
        pltpu.VMEM((16,), jnp.int32),
        pltpu.VMEM((16, D), jnp.float32),
        pltpu.VMEM((16, D), jnp.float32),
        pltpu.VMEM((_NB, D), jnp.float32),
        pltpu.VMEM((_NB, 16), jnp.float32),
        pltpu.VMEM((_NB, D), jnp.float32),
        pltpu.VMEM((40, 16), jnp.float32),
        pltpu.VMEM((40, 16), jnp.float32),
        pltpu.VMEM((_RPPAD,), jnp.int32),
        pltpu.SemaphoreType.DMA,
        pltpu.SemaphoreType.DMA,
    ],
)(_sc_edge_body)


def _pool_kernel(agg_ref, r_ref, oh_ref, out_ref):
    @pl.when(pl.program_id(0) == 0)
    def _():
        out_ref[:] = jnp.zeros_like(out_ref)
    h = agg_ref[:] + r_ref[:]
    out_ref[:] += lax.dot_general(oh_ref[:], h, (((1,), (0,)), ((), ())),
                                  preferred_element_type=jnp.float32,
                                  precision=_PREC)


_pool = pl.pallas_call(
    _pool_kernel, grid=(_GRID,),
    in_specs=[
        pl.BlockSpec((_MB, D), lambda i: (i, 0)),
        pl.BlockSpec((_MB, D), lambda i: (i, 0)),
        pl.BlockSpec((_G2, _MB), lambda i: (0, i)),
    ],
    out_specs=pl.BlockSpec((_G2, D), lambda i: (0, 0)),
    out_shape=jax.ShapeDtypeStruct((_G2, D), jnp.float32))


def _head_kernel(p_ref, lng_ref, lnb_ref, lw_ref, lb_ref, out_ref):
    pooled = p_ref[:]
    mu = pooled.mean(-1, keepdims=True)
    var = ((pooled - mu) ** 2).mean(-1, keepdims=True)
    z = ((pooled - mu) / jnp.sqrt(var + 1e-5) * lng_ref[:][None, :]
         + lnb_ref[:][None, :])
    emb = jnp.maximum(
        lax.dot_general(z, lw_ref[:], (((1,), (0,)), ((), ())),
                        preferred_element_type=jnp.float32,
                        precision=_PREC) + lb_ref[:][None, :], 0.0)
    emb_i = emb[:G]
    emb_j = emb[G:]
    dot = (emb_i * emb_j).sum(-1)
    ni = jnp.maximum(jnp.sqrt((emb_i * emb_i).sum(-1)), 1e-8)
    nj = jnp.maximum(jnp.sqrt((emb_j * emb_j).sum(-1)), 1e-8)
    out_ref[:] = dot / (ni * nj)


_head = pl.pallas_call(
    _head_kernel,
    out_shape=jax.ShapeDtypeStruct((G,), jnp.float32))


def _prep_edges(edge_index, edge_attr, node_off, edge_off):
    src = edge_index[0].astype(jnp.int32)
    dst = edge_index[1].astype(jnp.int32)
    order = jnp.argsort(dst)
    srcs = src[order] + node_off
    dsts = dst[order]
    eas = edge_attr[order]
    rp = jnp.searchsorted(dsts, jnp.arange(_NPAD // _NB + 1) * _NB,
                          side='left').astype(jnp.int32) + edge_off
    return srcs, dsts + node_off, eas, rp


def _prep_weights(Wq, Wk, Wv, Wr, We, origin, headeq):
    scale = np.float32(1.0 / np.sqrt(DH))
    wq_t = Wq[:, origin] * scale
    wk_t = Wk[:, origin]
    wv_t = Wv[:, origin]
    wr_t = Wr[:, origin]
    we_col = We[0][origin]
    wemask = we_col[:, None] * headeq
    wqwe = wq_t @ wemask
    wcat = jnp.concatenate([wq_t, wk_t, wv_t, wr_t, wqwe], axis=1)
    wet = we_col.reshape(40, 16)
    return wcat, wet


@jax.jit
def kernel(x_i, edge_index_i, edge_attr_i, batch_i, x_j, edge_index_j,
           edge_attr_j, batch_j, Wq0, Wk0, Wv0, Wr0, We0, WqR, WkR, WvR, WrR,
           WeR, ln_g, ln_b, lin_W, lin_b):
    origin = jnp.asarray(_ORIGIN_NP)
    headeq = jnp.asarray(_HEADEQ_NP)

    w0q = jnp.zeros((D, D), jnp.float32).at[:8].set(Wq0)
    w0k = jnp.zeros((D, D), jnp.float32).at[:8].set(Wk0)
    w0v = jnp.zeros((D, D), jnp.float32).at[:8].set(Wv0)
    w0r = jnp.zeros((D, D), jnp.float32).at[:8].set(Wr0)
    wcats = []
    wets = []
    wcat0, wet0 = _prep_weights(w0q, w0k, w0v, w0r, We0, origin, headeq)
    wcats.append(wcat0)
    wets.append(wet0)
    for l in range(L - 1):
        wc, wt = _prep_weights(WqR[l][origin, :], WkR[l][origin, :],
                               WvR[l][origin, :], WrR[l][origin, :],
                               WeR[l], origin, headeq)
        wcats.append(wc)
        wets.append(wt)
    wcats = jnp.stack(wcats)
    wets = jnp.stack(wets)
    flags = jnp.array([[0]] + [[1]] * (L - 1), jnp.int32)

    si, di, ei, rpi = _prep_edges(edge_index_i, edge_attr_i, 0, 0)
    sj, dj_, ej, rpj = _prep_edges(edge_index_j, edge_attr_j, _NPAD, E)
    srcs_u = jnp.concatenate([si, sj])
    dsts_u = jnp.concatenate([di, dj_])
    eas_u = jnp.concatenate([ei, ej])
    srcs_p = jnp.pad(srcs_u, (0, _EPAD - _EU))
    dsts_p = jnp.pad(dsts_u, (0, _EPAD - _EU))
    eas_p = jnp.pad(eas_u, (0, _EPAD - _EU))
    meta = jnp.concatenate([
        srcs_p.reshape(_MROWS, 16),
        dsts_p.reshape(_MROWS, 16),
        jnp.zeros((_MROWS, 16), jnp.int32)], axis=1)
    eam = jnp.concatenate([
        eas_p.reshape(_MROWS, 16),
        jnp.zeros((_MROWS, 16), jnp.float32)], axis=1)
    rp = jnp.concatenate([rpi[:-1], rpj])
    rp = jnp.pad(rp, (0, _RPPAD - (_NBLK + 1)))

    def one_hot(batch, off):
        b = jnp.pad(batch.astype(jnp.int32), (0, _NPAD - N),
                    constant_values=_G2 + 1)
        gids = jnp.arange(_G2, dtype=jnp.int32)
        return (gids[:, None] == (b[None, :] + off)).astype(jnp.float32)

    oh = jnp.concatenate([one_hot(batch_i, 0), one_hot(batch_j, G)], axis=1)

    act0 = jnp.zeros((_NU, D), jnp.float32)
    act0 = act0.at[:N, :8].set(x_i[:, 3:11])
    act0 = act0.at[_NPAD:_NPAD + N, :8].set(x_j[:, 3:11])
    r0 = jnp.zeros((_NU, D), jnp.float32)

    def step(carry, xs):
        agg, r = carry
        wcat, wet, flag = xs
        q, k, v, r_new, qwe = _proj(flag, agg, r, wcat)
        agg_new = _sc_edge(q, k, v, qwe, meta, eam, rp, wet)
        return (agg_new, r_new), 0

    (agg, r), _ = lax.scan(step, (act0, r0), (wcats, wets, flags))

    pooled = _pool(agg, r, oh)
    lng_t = ln_g[origin]
    lnb_t = ln_b[origin]
    lw_t = lin_W[origin, :]
    return _head(pooled, lng_t, lnb_t, lw_t, lin_b)
